# baseline (speedup 1.0000x reference)
import functools

import jax
import jax.numpy as jnp
from jax import lax
from jax.experimental import pallas as pl
from jax.experimental.pallas import tpu as pltpu
from jax.experimental.pallas import tpu_sc as plsc

_N_BLK = 2048
_GRP = 8
_LANE = 128


def _sc_info():
    info = plsc.get_sparse_core_info()
    return info.num_cores, info.num_subcores


def _make_sc_retile(V, D, n_groups):
    nc, ns = _sc_info()
    nw = nc * ns
    FC = 2 * _LANE
    NF = pl.cdiv(V, FC)
    GPF = FC // _GRP
    per_w = 4 * pl.cdiv(NF, 4 * nw)
    del n_groups
    mesh = plsc.VectorSubcoreMesh(core_axis_name="c", subcore_axis_name="s")

    @functools.partial(
        pl.kernel,
        mesh=mesh,
        out_type=jax.ShapeDtypeStruct((NF * GPF, _GRP * D), jnp.float32),
        compiler_params=pltpu.CompilerParams(needs_layout_passes=False),
        scratch_types=(
            [pltpu.VMEM((D, FC), jnp.float32)] * 4
            + [pltpu.VMEM((GPF, _GRP * D), jnp.float32)] * 4
            + [pltpu.SemaphoreType.DMA] * 8
        ),
    )
    def retile_kernel(emb_t_hbm, out_hbm, b0, b1, b2, b3, g0, g1, g2, g3,
                      si0, si1, si2, si3, so0, so1, so2, so3):
        bufs = [b0, b1, b2, b3]
        grps = [g0, g1, g2, g3]
        sis = [si0, si1, si2, si3]
        sos = [so0, so1, so2, so3]
        wid = lax.axis_index("s") * nc + lax.axis_index("c")
        base = wid * per_w
        iota16 = lax.iota(jnp.int32, 16)
        rowvs = [jnp.full((16,), g, jnp.int32) for g in range(GPF)]
        colvs = [(s * D) + iota16 for s in range(_GRP)]

        def shuffle(buf, grp):
            for c in range(FC):
                vals = plsc.load_gather(
                    buf, [iota16, jnp.full((16,), c, jnp.int32)])
                plsc.store_scatter(grp, [rowvs[c // _GRP], colvs[c % _GRP]],
                                   vals)

        def in_copy(f, buf, sem):
            return pltpu.make_async_copy(
                emb_t_hbm.at[:, pl.ds(f * FC, FC)], buf, sem)

        def out_copy(f, grp, sem):
            return pltpu.make_async_copy(
                grp, out_hbm.at[pl.ds(f * GPF, GPF)], sem)

        def slot(t, f, buf, grp, si, so):
            @pl.when(f < NF)
            def _():
                in_copy(f, buf, si).wait()

            @pl.when((t > 0) & (f - 4 < NF))
            def _():
                out_copy(f - 4, grp, so).wait()

            @pl.when(f < NF)
            def _():
                shuffle(buf, grp)
                out_copy(f, grp, so).start()

            @pl.when(((f + 4) < NF) & ((f + 4) < base + per_w))
            def _():
                in_copy(f + 4, buf, si).start()

        def body(t, _):
            for s in range(4):
                slot(t, base + 4 * t + s, bufs[s], grps[s], sis[s], sos[s])
            return None

        for s in range(4):
            @pl.when(base + s < NF)
            def _(s=s):
                in_copy(base + s, bufs[s], sis[s]).start()

        lax.fori_loop(0, per_w // 4, body, None)

        for s in range(4):
            last = base + per_w - 4 + s

            @pl.when(last < NF)
            def _(last=last, s=s):
                out_copy(last, grps[s], sos[s]).wait()

    return retile_kernel


def _make_sc_gather(G, D, B):
    nc, ns = _sc_info()
    nw = nc * ns
    assert B % (8 * nw) == 0
    b_per_w = B // nw
    n_chunks = b_per_w // 16
    mesh = plsc.VectorSubcoreMesh(core_axis_name="c", subcore_axis_name="s")

    @functools.partial(
        pl.kernel,
        mesh=mesh,
        out_type=jax.ShapeDtypeStruct((B, D), jnp.float32),
        compiler_params=pltpu.CompilerParams(needs_layout_passes=False),
        scratch_types=[
            pltpu.VMEM((b_per_w,), jnp.int32),
            pltpu.VMEM((b_per_w,), jnp.int32),
            pltpu.VMEM((b_per_w, _GRP * D), jnp.float32),
            pltpu.VMEM((b_per_w, D), jnp.float32),
            pltpu.SemaphoreType.DMA,
        ],
    )
    def gather_kernel(table_hbm, idx_hbm, out_hbm, idx_v, gidx_v, grp_v,
                      out_v, sem):
        wid = lax.axis_index("s") * nc + lax.axis_index("c")
        base = wid * b_per_w
        pltpu.sync_copy(idx_hbm.at[pl.ds(base, b_per_w)], idx_v)
        iota16 = lax.iota(jnp.int32, 16)
        for k in range(n_chunks):
            iv = idx_v[pl.ds(k * 16, 16)]
            gidx_v[pl.ds(k * 16, 16)] = lax.shift_right_logical(iv, 3)
        pltpu.async_copy(table_hbm.at[gidx_v], grp_v, sem).wait()
        for k in range(n_chunks):
            iv = idx_v[pl.ds(k * 16, 16)]
            colbase = (iv & (_GRP - 1)) * D
            rowids = iota16 + (k * 16)
            for j in range(D):
                vals = plsc.load_gather(grp_v, [rowids, colbase + j])
                jvec = jnp.full((16,), j, jnp.int32)
                plsc.store_scatter(out_v, [rowids, jvec], vals)
        pltpu.sync_copy(out_v, out_hbm.at[pl.ds(base, b_per_w)])

    return gather_kernel


def _mlp_body(rows_ref, w1_ref, b1_ref, w2t_ref, b2_ref, out_ref, h_ref):
    batch = out_ref.shape[1]
    emb_dim = rows_ref.shape[1]
    fan_in = rows_ref.shape[0] // batch

    @pl.when(pl.program_id(0) == 0)
    def _():
        acc = b1_ref[...]
        for j in range(fan_in):
            e_j = rows_ref[pl.ds(j * batch, batch), :]
            w1_j = w1_ref[:, pl.ds(j * emb_dim, emb_dim)]
            acc = acc + lax.dot_general(
                e_j, w1_j, (((1,), (1,)), ((), ())),
                preferred_element_type=jnp.float32)
        h_ref[...] = jnp.maximum(acc, 0.0)

    acc = lax.dot_general(
        w2t_ref[...], h_ref[...], (((0,), (1,)), ((), ())),
        preferred_element_type=jnp.float32)
    bias = lax.dot_general(
        b2_ref[...], jnp.ones((1, batch), jnp.float32),
        (((0,), (0,)), ((), ())), preferred_element_type=jnp.float32)
    out_ref[...] = acc + bias


def kernel(x, embed, W1, b1, W2, b2):
    batch, fan_in = x.shape
    vocab, hidden = W2.shape
    n_rows, emb_dim = embed.shape

    idx = x.T.reshape(-1).astype(jnp.int32)
    emb_t = embed.T

    n_groups = pl.cdiv(n_rows, _GRP)
    retile = _make_sc_retile(n_rows, emb_dim, n_groups)
    table_g = retile(emb_t)

    gather = _make_sc_gather(table_g.shape[0], emb_dim, idx.shape[0])
    rows = gather(table_g, idx)

    grid = pl.cdiv(vocab, _N_BLK)
    out_t = pl.pallas_call(
        _mlp_body,
        grid=(grid,),
        in_specs=[
            pl.BlockSpec((batch * fan_in, emb_dim), lambda i: (0, 0)),
            pl.BlockSpec(W1.shape, lambda i: (0, 0)),
            pl.BlockSpec((1, hidden), lambda i: (0, 0)),
            pl.BlockSpec((hidden, _N_BLK), lambda i: (0, i)),
            pl.BlockSpec((1, _N_BLK), lambda i: (0, i)),
        ],
        out_specs=pl.BlockSpec((_N_BLK, batch), lambda i: (i, 0)),
        out_shape=jax.ShapeDtypeStruct((vocab, batch), jnp.float32),
        scratch_shapes=[pltpu.VMEM((batch, hidden), jnp.float32)],
    )(rows, W1, b1.reshape(1, -1), W2.T, b2.reshape(1, -1))
    return out_t.T


# speedup vs baseline: 1.2173x; 1.2173x over previous
import functools

import jax
import jax.numpy as jnp
from jax import lax
from jax.experimental import pallas as pl
from jax.experimental.pallas import tpu as pltpu
from jax.experimental.pallas import tpu_sc as plsc

_N_BLK = 4096
_GRP = 8
_LANE = 128


def _sc_info():
    info = plsc.get_sparse_core_info()
    return info.num_cores, info.num_subcores


def _make_sc_retile(V, D, n_groups):
    nc, ns = _sc_info()
    nw = nc * ns
    NB = pl.cdiv(V, _LANE)
    GPB = _LANE // _GRP
    per_w = 2 * pl.cdiv(NB, 2 * nw)
    del n_groups
    mesh = plsc.VectorSubcoreMesh(core_axis_name="c", subcore_axis_name="s")

    @functools.partial(
        pl.kernel,
        mesh=mesh,
        out_type=jax.ShapeDtypeStruct((NB * GPB, _GRP * D), jnp.float32),
        compiler_params=pltpu.CompilerParams(needs_layout_passes=False),
        scratch_types=[
            pltpu.VMEM((D, _LANE), jnp.float32),
            pltpu.VMEM((D, _LANE), jnp.float32),
            pltpu.VMEM((GPB, _GRP * D), jnp.float32),
            pltpu.VMEM((GPB, _GRP * D), jnp.float32),
            pltpu.SemaphoreType.DMA,
            pltpu.SemaphoreType.DMA,
            pltpu.SemaphoreType.DMA,
            pltpu.SemaphoreType.DMA,
        ],
    )
    def retile_kernel(emb_t_hbm, out_hbm, buf0, buf1, grp0, grp1,
                      si0, si1, so0, so1):
        wid = lax.axis_index("s") * nc + lax.axis_index("c")
        base = wid * per_w
        iota16 = lax.iota(jnp.int32, 16)
        rowvs = [jnp.full((16,), g, jnp.int32) for g in range(GPB)]
        colvs = [(s * D) + iota16 for s in range(_GRP)]

        def shuffle(buf, grp):
            for c in range(_LANE):
                vals = plsc.load_gather(
                    buf, [iota16, jnp.full((16,), c, jnp.int32)])
                plsc.store_scatter(grp, [rowvs[c // _GRP], colvs[c % _GRP]],
                                   vals)

        def in_copy(blk, buf, sem):
            return pltpu.make_async_copy(
                emb_t_hbm.at[:, pl.ds(blk * _LANE, _LANE)], buf, sem)

        def out_copy(blk, grp, sem):
            return pltpu.make_async_copy(
                grp, out_hbm.at[pl.ds(blk * GPB, GPB)], sem)

        def halfstep(t, b, buf, grp, si, so):
            @pl.when(b < NB)
            def _():
                in_copy(b, buf, si).wait()

            @pl.when((t > 0) & (b - 2 < NB))
            def _():
                out_copy(b - 2, grp, so).wait()

            @pl.when(b < NB)
            def _():
                shuffle(buf, grp)
                out_copy(b, grp, so).start()

            @pl.when(((b + 2) < NB) & ((b + 2) < base + per_w))
            def _():
                in_copy(b + 2, buf, si).start()

        def body(t, _):
            b0 = base + 2 * t
            b1 = b0 + 1
            halfstep(t, b0, buf0, grp0, si0, so0)
            halfstep(t, b1, buf1, grp1, si1, so1)
            return None

        @pl.when(base < NB)
        def _():
            in_copy(base, buf0, si0).start()

        @pl.when(base + 1 < NB)
        def _():
            in_copy(base + 1, buf1, si1).start()

        lax.fori_loop(0, per_w // 2, body, None)

        last0 = base + per_w - 2
        last1 = base + per_w - 1

        @pl.when(last0 < NB)
        def _():
            out_copy(last0, grp0, so0).wait()

        @pl.when(last1 < NB)
        def _():
            out_copy(last1, grp1, so1).wait()

    return retile_kernel


def _make_sc_gather(G, D, B):
    nc, ns = _sc_info()
    nw = nc * ns
    assert B % (8 * nw) == 0
    b_per_w = B // nw
    n_chunks = b_per_w // 16
    mesh = plsc.VectorSubcoreMesh(core_axis_name="c", subcore_axis_name="s")

    @functools.partial(
        pl.kernel,
        mesh=mesh,
        out_type=jax.ShapeDtypeStruct((B, D), jnp.float32),
        compiler_params=pltpu.CompilerParams(needs_layout_passes=False),
        scratch_types=[
            pltpu.VMEM((b_per_w,), jnp.int32),
            pltpu.VMEM((b_per_w,), jnp.int32),
            pltpu.VMEM((b_per_w, _GRP * D), jnp.float32),
            pltpu.VMEM((b_per_w, D), jnp.float32),
            pltpu.SemaphoreType.DMA,
        ],
    )
    def gather_kernel(table_hbm, idx_hbm, out_hbm, idx_v, gidx_v, grp_v,
                      out_v, sem):
        wid = lax.axis_index("s") * nc + lax.axis_index("c")
        base = wid * b_per_w
        pltpu.sync_copy(idx_hbm.at[pl.ds(base, b_per_w)], idx_v)
        iota16 = lax.iota(jnp.int32, 16)
        for k in range(n_chunks):
            iv = idx_v[pl.ds(k * 16, 16)]
            gidx_v[pl.ds(k * 16, 16)] = lax.shift_right_logical(iv, 3)
        pltpu.async_copy(table_hbm.at[gidx_v], grp_v, sem).wait()
        for k in range(n_chunks):
            iv = idx_v[pl.ds(k * 16, 16)]
            colbase = (iv & (_GRP - 1)) * D
            rowids = iota16 + (k * 16)
            for j in range(D):
                vals = plsc.load_gather(grp_v, [rowids, colbase + j])
                jvec = jnp.full((16,), j, jnp.int32)
                plsc.store_scatter(out_v, [rowids, jvec], vals)
        pltpu.sync_copy(out_v, out_hbm.at[pl.ds(base, b_per_w)])

    return gather_kernel


def _mlp_body(rows_ref, w1_ref, b1_ref, w2t_ref, b2_ref, out_ref, h_ref):
    batch = out_ref.shape[1]
    emb_dim = rows_ref.shape[1]
    fan_in = rows_ref.shape[0] // batch

    @pl.when(pl.program_id(0) == 0)
    def _():
        acc = b1_ref[...]
        for j in range(fan_in):
            e_j = rows_ref[pl.ds(j * batch, batch), :]
            w1_j = w1_ref[:, pl.ds(j * emb_dim, emb_dim)]
            acc = acc + lax.dot_general(
                e_j, w1_j, (((1,), (1,)), ((), ())),
                preferred_element_type=jnp.float32)
        h_ref[...] = jnp.maximum(acc, 0.0)

    acc = lax.dot_general(
        w2t_ref[...], h_ref[...], (((0,), (1,)), ((), ())),
        preferred_element_type=jnp.float32)
    bias = lax.dot_general(
        b2_ref[...], jnp.ones((1, batch), jnp.float32),
        (((0,), (0,)), ((), ())), preferred_element_type=jnp.float32)
    out_ref[...] = acc + bias


def kernel(x, embed, W1, b1, W2, b2):
    batch, fan_in = x.shape
    vocab, hidden = W2.shape
    n_rows, emb_dim = embed.shape

    idx = x.T.reshape(-1).astype(jnp.int32)
    emb_t = embed.T

    n_groups = pl.cdiv(n_rows, _GRP)
    retile = _make_sc_retile(n_rows, emb_dim, n_groups)
    table_g = retile(emb_t)

    gather = _make_sc_gather(table_g.shape[0], emb_dim, idx.shape[0])
    rows = gather(table_g, idx)

    grid = pl.cdiv(vocab, _N_BLK)
    out_t = pl.pallas_call(
        _mlp_body,
        grid=(grid,),
        in_specs=[
            pl.BlockSpec((batch * fan_in, emb_dim), lambda i: (0, 0)),
            pl.BlockSpec(W1.shape, lambda i: (0, 0)),
            pl.BlockSpec((1, hidden), lambda i: (0, 0)),
            pl.BlockSpec((hidden, _N_BLK), lambda i: (0, i)),
            pl.BlockSpec((1, _N_BLK), lambda i: (0, i)),
        ],
        out_specs=pl.BlockSpec((_N_BLK, batch), lambda i: (i, 0)),
        out_shape=jax.ShapeDtypeStruct((vocab, batch), jnp.float32),
        scratch_shapes=[pltpu.VMEM((batch, hidden), jnp.float32)],
    )(rows, W1, b1.reshape(1, -1), W2.T, b2.reshape(1, -1))
    return out_t.T
